# baseline (speedup 1.0000x reference)
import jax
import jax.numpy as jnp
from jax.experimental import pallas as pl
from jax.experimental.pallas import tpu as pltpu

_CHUNK = 256


def _router_kernel(task_id_ref, mode_id_ref, x_ref, w_ref, tb_ref, mb_ref,
                   idx_ref, wgt_ref, acc_ref):
    c = pl.program_id(0)
    nc = pl.num_programs(0)

    @pl.when(c == 0)
    def _init():
        acc_ref[:] = jnp.zeros_like(acc_ref)

    acc_ref[:] += jnp.sum(x_ref[:], axis=1)

    @pl.when(c == nc - 1)
    def _finish():
        B, D = acc_ref.shape
        E = w_ref.shape[0]
        T = tb_ref.shape[0]
        M = mb_ref.shape[0]
        S = nc * x_ref.shape[1]

        pooled = acc_ref[:] * (1.0 / S)
        logits = jax.lax.dot_general(
            pooled, w_ref[:], (((1,), (1,)), ((), ())),
            preferred_element_type=jnp.float32)

        t_iota = jax.lax.broadcasted_iota(jnp.int32, (1, T), 1)
        m_iota = jax.lax.broadcasted_iota(jnp.int32, (1, M), 1)
        oh_t = jnp.concatenate(
            [(t_iota == task_id_ref[b]).astype(jnp.float32) for b in range(B)],
            axis=0)
        oh_m = jnp.concatenate(
            [(m_iota == mode_id_ref[b]).astype(jnp.float32) for b in range(B)],
            axis=0)
        logits = logits + oh_t @ tb_ref[:] + oh_m @ mb_ref[:]

        e_iota = jax.lax.broadcasted_iota(jnp.int32, (B, E), 1)
        m1 = jnp.max(logits, axis=1, keepdims=True)
        i1 = jnp.min(jnp.where(logits == m1, e_iota, E), axis=1, keepdims=True)
        masked = jnp.where(e_iota == i1, -jnp.inf, logits)
        m2 = jnp.max(masked, axis=1, keepdims=True)
        i2 = jnp.min(jnp.where(masked == m2, e_iota, E), axis=1, keepdims=True)

        idx_ref[:] = jnp.concatenate([i1, i2], axis=1)
        r = jnp.exp(m2 - m1)
        w1 = 1.0 / (1.0 + r)
        wgt_ref[:] = jnp.concatenate([w1, 1.0 - w1], axis=1)


@jax.jit
def _impl(x, task_id, mode_id, W, task_bias, mode_bias):
    B, S, D = x.shape
    E = W.shape[0]
    chunk = _CHUNK if S % _CHUNK == 0 else S
    nc = S // chunk

    idx, wgt = pl.pallas_call(
        _router_kernel,
        grid=(nc,),
        in_specs=[
            pl.BlockSpec(memory_space=pltpu.SMEM),
            pl.BlockSpec(memory_space=pltpu.SMEM),
            pl.BlockSpec((B, chunk, D), lambda c: (0, c, 0)),
            pl.BlockSpec(W.shape, lambda c: (0, 0)),
            pl.BlockSpec(task_bias.shape, lambda c: (0, 0)),
            pl.BlockSpec(mode_bias.shape, lambda c: (0, 0)),
        ],
        out_specs=[
            pl.BlockSpec((B, 2), lambda c: (0, 0)),
            pl.BlockSpec((B, 2), lambda c: (0, 0)),
        ],
        out_shape=[
            jax.ShapeDtypeStruct((B, 2), jnp.int32),
            jax.ShapeDtypeStruct((B, 2), jnp.float32),
        ],
        scratch_shapes=[pltpu.VMEM((B, D), jnp.float32)],
        compiler_params=pltpu.CompilerParams(
            dimension_semantics=("arbitrary",)),
    )(task_id.astype(jnp.int32), mode_id.astype(jnp.int32),
      x, W, task_bias, mode_bias)
    return idx, wgt


def kernel(x, task_id, mode_id, W, task_bias, mode_bias):
    return _impl(x, task_id, mode_id, W, task_bias, mode_bias)
